# baseline (speedup 1.0000x reference)
import functools

import jax
import jax.numpy as jnp
from jax import lax
from jax.experimental import pallas as pl
from jax.experimental.pallas import tpu as pltpu
from jax.experimental.pallas import tpu_sc as plsc

_B, _C, _H, _W = 128, 3, 256, 256
_R = 4
_TPAD = 64
_D = 128

_NC, _NS = 2, 16
_NWORK = 16
_BPW = _B // _NWORK


def _sc_gather(table, t):
    mesh = plsc.VectorSubcoreMesh(core_axis_name="c", subcore_axis_name="s")

    @functools.partial(
        pl.kernel,
        mesh=mesh,
        out_type=jax.ShapeDtypeStruct((_B, _D), jnp.float32),
        scratch_types=[
            pltpu.VMEM((_BPW,), jnp.int32),
            pltpu.VMEM((_BPW, _D), jnp.float32),
            pltpu.SemaphoreType.DMA,
        ],
    )
    def k(table_hbm, t_hbm, out_hbm, idx_v, rows_v, sem):
        wid = lax.axis_index("s") * _NC + lax.axis_index("c")

        @pl.when(wid < _NWORK)
        def _():
            base = wid * _BPW
            pltpu.sync_copy(t_hbm.at[pl.ds(base, _BPW)], idx_v)
            pltpu.async_copy(table_hbm.at[idx_v], rows_v, sem).wait()
            pltpu.sync_copy(rows_v, out_hbm.at[pl.ds(base, _BPW)])

    return k(table, t)


def _qsample_body(a_ref, b_ref, x_ref, n_ref, o_ref):
    i = pl.program_id(0)
    for r in range(_R):
        a = a_ref[i * _R + r]
        b = b_ref[i * _R + r]
        o_ref[r] = a * x_ref[r] + b * n_ref[r]


def kernel(x_start, t, noise, sqrt_alphas_cumprod, sqrt_one_minus_alphas_cumprod):
    table = jnp.zeros((_TPAD, _D), jnp.float32)
    table = table.at[: sqrt_alphas_cumprod.shape[0], 0].set(sqrt_alphas_cumprod)
    table = table.at[: sqrt_one_minus_alphas_cumprod.shape[0], 1].set(
        sqrt_one_minus_alphas_cumprod
    )

    coeffs = _sc_gather(table, t)
    a = coeffs[:, 0]
    b = coeffs[:, 1]

    grid_spec = pltpu.PrefetchScalarGridSpec(
        num_scalar_prefetch=2,
        grid=(_B // _R,),
        in_specs=[
            pl.BlockSpec((_R, _C, _H, _W), lambda i, *_: (i, 0, 0, 0)),
            pl.BlockSpec((_R, _C, _H, _W), lambda i, *_: (i, 0, 0, 0)),
        ],
        out_specs=pl.BlockSpec((_R, _C, _H, _W), lambda i, *_: (i, 0, 0, 0)),
    )

    return pl.pallas_call(
        _qsample_body,
        grid_spec=grid_spec,
        out_shape=jax.ShapeDtypeStruct((_B, _C, _H, _W), jnp.float32),
    )(a, b, x_start, noise)


# speedup vs baseline: 1.2533x; 1.2533x over previous
import jax
import jax.numpy as jnp
from jax.experimental import pallas as pl
from jax.experimental.pallas import tpu as pltpu

_B, _C, _H, _W = 128, 3, 256, 256
_CHW = _C * _H * _W
_LANES = 128
_SUB = _CHW // _LANES
_R = 4


def _qsample_body(t_ref, sac_ref, som_ref, x_ref, n_ref, o_ref):
    i = pl.program_id(0)
    for r in range(_R):
        tt = t_ref[i * _R + r]
        a = sac_ref[tt]
        b = som_ref[tt]
        o_ref[r] = a * x_ref[r] + b * n_ref[r]


def kernel(x_start, t, noise, sqrt_alphas_cumprod, sqrt_one_minus_alphas_cumprod):
    grid_spec = pltpu.PrefetchScalarGridSpec(
        num_scalar_prefetch=3,
        grid=(_B // _R,),
        in_specs=[
            pl.BlockSpec((_R, _C, _H, _W), lambda i, *_: (i, 0, 0, 0)),
            pl.BlockSpec((_R, _C, _H, _W), lambda i, *_: (i, 0, 0, 0)),
        ],
        out_specs=pl.BlockSpec((_R, _C, _H, _W), lambda i, *_: (i, 0, 0, 0)),
    )

    return pl.pallas_call(
        _qsample_body,
        grid_spec=grid_spec,
        out_shape=jax.ShapeDtypeStruct((_B, _C, _H, _W), jnp.float32),
        compiler_params=pltpu.CompilerParams(dimension_semantics=("parallel",)),
    )(t, sqrt_alphas_cumprod, sqrt_one_minus_alphas_cumprod, x_start, noise)
